# baseline (speedup 1.0000x reference)
import functools

import jax
import jax.numpy as jnp
from jax import lax
from jax.experimental import pallas as pl
from jax.experimental.pallas import tpu as pltpu
from jax.experimental.pallas import tpu_sc as plsc

N = 204800
V = 100000
C = 128
NC = 2
NS = 16
NW = NC * NS
ROWS_PER_W = N // NW
CH = 128
NCH = ROWS_PER_W // CH
NBUF = 5


@functools.cache
def _make_sc_gather():
    mesh = plsc.VectorSubcoreMesh(core_axis_name="c", subcore_axis_name="s")

    @functools.partial(
        pl.kernel,
        mesh=mesh,
        out_type=jax.ShapeDtypeStruct((N, C), jnp.float32),
        scratch_types=[
            pltpu.VMEM((NCH, CH), jnp.int32),
            pltpu.VMEM((NBUF, CH, C), jnp.float32),
        ]
        + [pltpu.SemaphoreType.DMA] * (2 * NBUF),
    )
    def _sc_gather(idx_hbm, table_hbm, out_hbm, idx_v, rows_v, *sems):
        sem_g, sem_o = sems[:NBUF], sems[NBUF:]
        wid = lax.axis_index("s") * NC + lax.axis_index("c")
        base = wid * ROWS_PER_W

        pltpu.sync_copy(idx_hbm.at[wid], idx_v)

        def start_gather(chunk, b):
            pltpu.async_copy(table_hbm.at[idx_v.at[chunk]], rows_v.at[b], sem_g[b])

        def wait_gather(b):
            pltpu.make_async_copy(
                out_hbm.at[pl.ds(0, CH)], rows_v.at[b], sem_g[b]
            ).wait()

        def start_out(chunk, b):
            off = base + chunk * CH
            pltpu.async_copy(rows_v.at[b], out_hbm.at[pl.ds(off, CH)], sem_o[b])

        def wait_out(b):
            pltpu.make_async_copy(
                rows_v.at[b], out_hbm.at[pl.ds(0, CH)], sem_o[b]
            ).wait()

        for b in range(NBUF):
            start_gather(b, b)

        def group(g, carry):
            i0 = g * NBUF
            for b in range(NBUF):
                chunk = i0 + b
                wait_gather(b)
                start_out(chunk, b)

                @pl.when(chunk + NBUF < NCH)
                def _():
                    wait_out(b)
                    start_gather(chunk + NBUF, b)

            return carry

        lax.fori_loop(0, NCH // NBUF, group, 0)
        for b in range(NBUF):
            wait_out(b)

    return _sc_gather


@functools.cache
def _make_sc_pick():
    mesh = plsc.VectorSubcoreMesh(core_axis_name="c", subcore_axis_name="s")
    L = 16

    @functools.partial(
        pl.kernel,
        mesh=mesh,
        out_type=(
            jax.ShapeDtypeStruct((N,), jnp.float32),
            jax.ShapeDtypeStruct((N,), jnp.float32),
        ),
        scratch_types=[
            pltpu.VMEM((NCH, CH), jnp.int32),
            pltpu.VMEM((NCH, CH), jnp.int32),
            pltpu.VMEM((NCH, CH), jnp.int32),
            pltpu.VMEM((NCH * CH,), jnp.float32),
            pltpu.VMEM((NCH * CH,), jnp.float32),
            pltpu.SemaphoreType.DMA,
        ],
    )
    def _sc_pick(
        idx_hbm, tgt_hbm, lse_hbm, tablef_hbm, stok_hbm, ptok_hbm,
        idx_v, tgt_v, pidx_v, sv, pv, sem_p,
    ):
        wid = lax.axis_index("s") * NC + lax.axis_index("c")
        base = wid * ROWS_PER_W

        pltpu.sync_copy(idx_hbm.at[wid], idx_v)
        pltpu.sync_copy(tgt_hbm.at[wid], tgt_v)

        def chunk_body(chunk, carry):
            for j in range(CH // L):
                sl = pl.ds(j * L, L)
                pidx_v[chunk, sl] = idx_v[chunk, sl] * C + tgt_v[chunk, sl]
            pltpu.async_copy(
                lse_hbm.at[idx_v.at[chunk]], sv.at[pl.ds(chunk * CH, CH)], sem_p
            )
            pltpu.async_copy(
                tablef_hbm.at[pidx_v.at[chunk]], pv.at[pl.ds(chunk * CH, CH)], sem_p
            )
            return carry

        lax.fori_loop(0, NCH, chunk_body, 0)
        pltpu.make_async_copy(stok_hbm.at[pl.ds(0, NCH * CH)], sv, sem_p).wait()
        pltpu.make_async_copy(ptok_hbm.at[pl.ds(0, NCH * CH)], pv, sem_p).wait()
        pltpu.sync_copy(sv, stok_hbm.at[pl.ds(base, ROWS_PER_W)])
        pltpu.sync_copy(pv, ptok_hbm.at[pl.ds(base, ROWS_PER_W)])

    return _sc_pick


RBL = 2048
GL = -(-V // RBL)
VP = GL * RBL
SW = 8


def _lse_body(x_ref, lse_ref, flat_ref):
    x = x_ref[...]
    e = jnp.exp(x)
    s = lax.dot_general(
        e,
        jnp.ones((C, SW), jnp.float32),
        (((1,), (0,)), ((), ())),
        preferred_element_type=jnp.float32,
    )
    lse_ref[...] = jnp.log(s[:, 0])
    flat_ref[...] = x.reshape(RBL * C)


_tc_lse = pl.pallas_call(
    _lse_body,
    grid=(GL,),
    in_specs=[pl.BlockSpec((RBL, C), lambda i: (i, 0))],
    out_specs=[
        pl.BlockSpec((RBL,), lambda i: (i,)),
        pl.BlockSpec((RBL * C,), lambda i: (i,)),
    ],
    out_shape=[
        jax.ShapeDtypeStruct((VP,), jnp.float32),
        jax.ShapeDtypeStruct((VP * C,), jnp.float32),
    ],
)


def _final_body(s_ref, p_ref, out_ref):
    out_ref[0, 0] = (jnp.sum(s_ref[...]) - jnp.sum(p_ref[...])) / N


_tc_final = pl.pallas_call(
    _final_body,
    in_specs=[
        pl.BlockSpec((N // C, C), lambda: (0, 0)),
        pl.BlockSpec((N // C, C), lambda: (0, 0)),
    ],
    out_specs=pl.BlockSpec((1, 1), lambda: (0, 0), memory_space=pltpu.SMEM),
    out_shape=jax.ShapeDtypeStruct((1, 1), jnp.float32),
)


def kernel(idx, targets, table):
    idx_w = idx.reshape(NW, NCH, CH).astype(jnp.int32)
    tgt_w = targets.reshape(NW, NCH, CH).astype(jnp.int32)
    lse_all, tablef = _tc_lse(table)
    logits = _make_sc_gather()(idx_w, table)
    stok, ptok = _make_sc_pick()(idx_w, tgt_w, lse_all, tablef)
    loss = _tc_final(stok.reshape(N // C, C), ptok.reshape(N // C, C))[0, 0]
    return logits, loss


# speedup vs baseline: 1.2241x; 1.2241x over previous
import functools

import jax
import jax.numpy as jnp
from jax import lax
from jax.experimental import pallas as pl
from jax.experimental.pallas import tpu as pltpu
from jax.experimental.pallas import tpu_sc as plsc

N = 204800
V = 100000
C = 128
NC = 2
NS = 16
NW = NC * NS
ROWS_PER_W = N // NW
CH = 128
NCH = ROWS_PER_W // CH
NBUF = 5


@functools.cache
def _make_sc_gather():
    mesh = plsc.VectorSubcoreMesh(core_axis_name="c", subcore_axis_name="s")

    @functools.partial(
        pl.kernel,
        mesh=mesh,
        out_type=jax.ShapeDtypeStruct((N, C), jnp.float32),
        scratch_types=[
            pltpu.VMEM((NCH, CH), jnp.int32),
            pltpu.VMEM((NBUF, CH, C), jnp.float32),
        ]
        + [pltpu.SemaphoreType.DMA] * (2 * NBUF),
    )
    def _sc_gather(idx_hbm, table_hbm, out_hbm, idx_v, rows_v, *sems):
        sem_g, sem_o = sems[:NBUF], sems[NBUF:]
        wid = lax.axis_index("s") * NC + lax.axis_index("c")
        base = wid * ROWS_PER_W

        pltpu.sync_copy(idx_hbm.at[wid], idx_v)

        def start_gather(chunk, b):
            pltpu.async_copy(table_hbm.at[idx_v.at[chunk]], rows_v.at[b], sem_g[b])

        def wait_gather(b):
            pltpu.make_async_copy(
                table_hbm.at[pl.ds(0, CH)], rows_v.at[b], sem_g[b]
            ).wait()

        def start_out(chunk, b):
            off = base + chunk * CH
            pltpu.async_copy(rows_v.at[b], out_hbm.at[pl.ds(off, CH)], sem_o[b])

        def wait_out(b):
            pltpu.make_async_copy(
                rows_v.at[b], out_hbm.at[pl.ds(0, CH)], sem_o[b]
            ).wait()

        for b in range(NBUF):
            start_gather(b, b)

        def group(g, carry):
            i0 = g * NBUF
            for b in range(NBUF):
                chunk = i0 + b
                wait_gather(b)
                start_out(chunk, b)

                @pl.when(chunk + NBUF < NCH)
                def _():
                    wait_out(b)
                    start_gather(chunk + NBUF, b)

            return carry

        lax.fori_loop(0, NCH // NBUF, group, 0)
        for b in range(NBUF):
            wait_out(b)

    return _sc_gather


@functools.cache
def _make_sc_pick():
    mesh = plsc.VectorSubcoreMesh(core_axis_name="c", subcore_axis_name="s")
    L = 16

    @functools.partial(
        pl.kernel,
        mesh=mesh,
        out_type=(
            jax.ShapeDtypeStruct((N,), jnp.float32),
            jax.ShapeDtypeStruct((N,), jnp.float32),
        ),
        scratch_types=[
            pltpu.VMEM((NCH, CH), jnp.int32),
            pltpu.VMEM((NCH, CH), jnp.int32),
            pltpu.VMEM((NCH, CH), jnp.int32),
            pltpu.VMEM((NCH * CH,), jnp.float32),
            pltpu.VMEM((NCH * CH,), jnp.float32),
            pltpu.SemaphoreType.DMA,
        ],
    )
    def _sc_pick(
        idx_hbm, tgt_hbm, lse_hbm, tablef_hbm, stok_hbm, ptok_hbm,
        idx_v, tgt_v, pidx_v, sv, pv, sem_p,
    ):
        wid = lax.axis_index("s") * NC + lax.axis_index("c")
        base = wid * ROWS_PER_W

        pltpu.sync_copy(idx_hbm.at[wid], idx_v)
        pltpu.sync_copy(tgt_hbm.at[wid], tgt_v)

        def chunk_body(chunk, carry):
            for j in range(CH // L):
                sl = pl.ds(j * L, L)
                pidx_v[chunk, sl] = idx_v[chunk, sl] * C + tgt_v[chunk, sl]
            pltpu.async_copy(
                lse_hbm.at[idx_v.at[chunk]], sv.at[pl.ds(chunk * CH, CH)], sem_p
            )
            pltpu.async_copy(
                tablef_hbm.at[pidx_v.at[chunk]], pv.at[pl.ds(chunk * CH, CH)], sem_p
            )
            return carry

        lax.fori_loop(0, NCH, chunk_body, 0)
        pltpu.make_async_copy(stok_hbm.at[pl.ds(0, NCH * CH)], sv, sem_p).wait()
        pltpu.make_async_copy(ptok_hbm.at[pl.ds(0, NCH * CH)], pv, sem_p).wait()
        pltpu.sync_copy(sv, stok_hbm.at[pl.ds(base, ROWS_PER_W)])
        pltpu.sync_copy(pv, ptok_hbm.at[pl.ds(base, ROWS_PER_W)])

    return _sc_pick


RBL = 2048
GL = -(-V // RBL)
VP = GL * RBL
SW = 8


def _lse_body(x_ref, lse_ref, flat_ref):
    x = x_ref[...]
    e = jnp.exp(x)
    s = lax.dot_general(
        e,
        jnp.ones((C, SW), jnp.float32),
        (((1,), (0,)), ((), ())),
        preferred_element_type=jnp.float32,
    )
    lse_ref[...] = jnp.log(s[:, 0])
    flat_ref[...] = x.reshape(RBL * C)


_tc_lse = pl.pallas_call(
    _lse_body,
    grid=(GL,),
    in_specs=[pl.BlockSpec((RBL, C), lambda i: (i, 0))],
    out_specs=[
        pl.BlockSpec((RBL,), lambda i: (i,)),
        pl.BlockSpec((RBL * C,), lambda i: (i,)),
    ],
    out_shape=[
        jax.ShapeDtypeStruct((VP,), jnp.float32),
        jax.ShapeDtypeStruct((VP * C,), jnp.float32),
    ],
)


def _final_body(s_ref, p_ref, out_ref):
    out_ref[0, 0] = (jnp.sum(s_ref[...]) - jnp.sum(p_ref[...])) / N


_tc_final = pl.pallas_call(
    _final_body,
    in_specs=[
        pl.BlockSpec((N // C, C), lambda: (0, 0)),
        pl.BlockSpec((N // C, C), lambda: (0, 0)),
    ],
    out_specs=pl.BlockSpec((1, 1), lambda: (0, 0), memory_space=pltpu.SMEM),
    out_shape=jax.ShapeDtypeStruct((1, 1), jnp.float32),
)


def kernel(idx, targets, table):
    idx_w = idx.reshape(NW, NCH, CH).astype(jnp.int32)
    tgt_w = targets.reshape(NW, NCH, CH).astype(jnp.int32)
    lse_all, tablef = _tc_lse(table)
    logits = _make_sc_gather()(idx_w, table)
    lse_b, tablef_b, logits_b = lax.optimization_barrier(
        (lse_all, tablef, logits)
    )
    stok, ptok = _make_sc_pick()(idx_w, tgt_w, lse_b, tablef_b)
    loss = _tc_final(stok.reshape(N // C, C), ptok.reshape(N // C, C))[0, 0]
    return logits_b, loss


# speedup vs baseline: 1.2617x; 1.0307x over previous
import functools

import jax
import jax.numpy as jnp
from jax import lax
from jax.experimental import pallas as pl
from jax.experimental.pallas import tpu as pltpu
from jax.experimental.pallas import tpu_sc as plsc

N = 204800
V = 100000
C = 128
NC = 2
NS = 16
NW = NC * NS
ROWS_PER_W = N // NW
CH = 128
NCH = ROWS_PER_W // CH
NBUF = 5


ZCH = 12544


@functools.cache
def _make_sc_gather():
    mesh = plsc.VectorSubcoreMesh(core_axis_name="c", subcore_axis_name="s")
    L = 16

    @functools.partial(
        pl.kernel,
        mesh=mesh,
        out_type=(
            jax.ShapeDtypeStruct((N, C), jnp.float32),
            jax.ShapeDtypeStruct((NC, VP), jnp.float32),
        ),
        scratch_types=[
            pltpu.VMEM((NCH, CH), jnp.int32),
            pltpu.VMEM((NBUF, CH, C), jnp.float32),
            pltpu.VMEM((ZCH,), jnp.float32),
            pltpu.VMEM((CH,), jnp.float32),
            pltpu.VMEM_SHARED((VP,), jnp.float32),
        ]
        + [pltpu.SemaphoreType.DMA] * (2 * NBUF),
    )
    def _sc_gather(
        idx_hbm, table_hbm, out_hbm, cnt_hbm,
        idx_v, rows_v, zbuf, ones_v, shared_cnt, *sems,
    ):
        sem_g, sem_o = sems[:NBUF], sems[NBUF:]
        sid = lax.axis_index("s")
        cid = lax.axis_index("c")
        wid = sid * NC + cid
        base = wid * ROWS_PER_W

        pltpu.sync_copy(idx_hbm.at[wid], idx_v)

        def fill(i, carry):
            for j in range(8):
                zbuf[pl.ds(i * (8 * L) + j * L, L)] = jnp.zeros((L,), jnp.float32)
            return carry

        @pl.when(sid == 0)
        def _():
            lax.fori_loop(0, ZCH // (8 * L), fill, 0)
            for k in range(VP // ZCH):
                pltpu.sync_copy(zbuf, shared_cnt.at[pl.ds(k * ZCH, ZCH)])

        for j in range(CH // L):
            ones_v[pl.ds(j * L, L)] = jnp.ones((L,), jnp.float32)
        plsc.subcore_barrier()

        def start_gather(chunk, b):
            pltpu.async_copy(table_hbm.at[idx_v.at[chunk]], rows_v.at[b], sem_g[b])

        def wait_gather(b):
            pltpu.make_async_copy(
                table_hbm.at[pl.ds(0, CH)], rows_v.at[b], sem_g[b]
            ).wait()

        def start_out(chunk, b):
            off = base + chunk * CH
            pltpu.async_copy(rows_v.at[b], out_hbm.at[pl.ds(off, CH)], sem_o[b])

        def wait_out(b):
            pltpu.make_async_copy(
                rows_v.at[b], out_hbm.at[pl.ds(0, CH)], sem_o[b]
            ).wait()

        for b in range(NBUF):
            start_gather(b, b)

        def group(g, carry):
            i0 = g * NBUF
            for b in range(NBUF):
                chunk = i0 + b
                wait_gather(b)
                start_out(chunk, b)
                pltpu.sync_copy(ones_v, shared_cnt.at[idx_v.at[chunk]], add=True)

                @pl.when(chunk + NBUF < NCH)
                def _():
                    wait_out(b)
                    start_gather(chunk + NBUF, b)

            return carry

        lax.fori_loop(0, NCH // NBUF, group, 0)
        plsc.subcore_barrier()

        @pl.when(sid == 0)
        def _():
            pltpu.sync_copy(shared_cnt, cnt_hbm.at[cid])

        for b in range(NBUF):
            wait_out(b)

    return _sc_gather


@functools.cache
def _make_sc_pick():
    mesh = plsc.VectorSubcoreMesh(core_axis_name="c", subcore_axis_name="s")
    L = 16

    @functools.partial(
        pl.kernel,
        mesh=mesh,
        out_type=jax.ShapeDtypeStruct((N,), jnp.float32),
        scratch_types=[
            pltpu.VMEM((NCH, CH), jnp.int32),
            pltpu.VMEM((NCH, CH), jnp.int32),
            pltpu.VMEM((NCH, CH), jnp.int32),
            pltpu.VMEM((NCH * CH,), jnp.float32),
            pltpu.SemaphoreType.DMA,
        ],
    )
    def _sc_pick(
        idx_hbm, tgt_hbm, tablef_hbm, ptok_hbm,
        idx_v, tgt_v, pidx_v, pv, sem_p,
    ):
        wid = lax.axis_index("s") * NC + lax.axis_index("c")
        base = wid * ROWS_PER_W

        pltpu.sync_copy(idx_hbm.at[wid], idx_v)
        pltpu.sync_copy(tgt_hbm.at[wid], tgt_v)

        def chunk_body(chunk, carry):
            for j in range(CH // L):
                sl = pl.ds(j * L, L)
                pidx_v[chunk, sl] = idx_v[chunk, sl] * C + tgt_v[chunk, sl]
            pltpu.async_copy(
                tablef_hbm.at[pidx_v.at[chunk]], pv.at[pl.ds(chunk * CH, CH)], sem_p
            )
            return carry

        lax.fori_loop(0, NCH, chunk_body, 0)
        pltpu.make_async_copy(ptok_hbm.at[pl.ds(0, NCH * CH)], pv, sem_p).wait()
        pltpu.sync_copy(pv, ptok_hbm.at[pl.ds(base, ROWS_PER_W)])

    return _sc_pick


RBL = 2048
GL = -(-V // RBL)
VP = GL * RBL
SW = 8


def _lse_body(x_ref, lse_ref, flat_ref):
    x = x_ref[...]
    e = jnp.exp(x)
    s = lax.dot_general(
        e,
        jnp.ones((C, SW), jnp.float32),
        (((1,), (0,)), ((), ())),
        preferred_element_type=jnp.float32,
    )
    lse_ref[...] = jnp.log(s[:, 0])
    flat_ref[...] = x.reshape(RBL * C)


_tc_lse = pl.pallas_call(
    _lse_body,
    grid=(GL,),
    in_specs=[pl.BlockSpec((RBL, C), lambda i: (i, 0))],
    out_specs=[
        pl.BlockSpec((RBL,), lambda i: (i,)),
        pl.BlockSpec((RBL * C,), lambda i: (i,)),
    ],
    out_shape=[
        jax.ShapeDtypeStruct((VP,), jnp.float32),
        jax.ShapeDtypeStruct((VP * C,), jnp.float32),
    ],
)


def _final_body(lse_ref, cnt_ref, p_ref, out_ref):
    c = cnt_ref[: VP // C] + cnt_ref[VP // C :]
    lse = lse_ref[...]
    lse_total = jnp.sum(jnp.where(c > 0.0, c * lse, 0.0))
    out_ref[0, 0] = (lse_total - jnp.sum(p_ref[...])) / N


_tc_final = pl.pallas_call(
    _final_body,
    in_specs=[
        pl.BlockSpec((VP // C, C), lambda: (0, 0)),
        pl.BlockSpec((NC * VP // C, C), lambda: (0, 0)),
        pl.BlockSpec((N // C, C), lambda: (0, 0)),
    ],
    out_specs=pl.BlockSpec((1, 1), lambda: (0, 0), memory_space=pltpu.SMEM),
    out_shape=jax.ShapeDtypeStruct((1, 1), jnp.float32),
)


def kernel(idx, targets, table):
    idx_w = idx.reshape(NW, NCH, CH).astype(jnp.int32)
    tgt_w = targets.reshape(NW, NCH, CH).astype(jnp.int32)
    lse_all, tablef = _tc_lse(table)
    logits, cnt = _make_sc_gather()(idx_w, table)
    tablef_b, logits_b = lax.optimization_barrier((tablef, logits))
    ptok = _make_sc_pick()(idx_w, tgt_w, tablef_b)
    loss = _tc_final(
        lse_all.reshape(VP // C, C),
        cnt.reshape(NC * VP // C, C),
        ptok.reshape(N // C, C),
    )[0, 0]
    return logits_b, loss
